# baseline (device time: 28682 ns/iter reference)
import jax
import jax.numpy as jnp
from jax import lax
from jax.experimental import pallas as pl
from jax.experimental.pallas import tpu as pltpu

N_DEV = 4
N_CHUNKS = 8


def kernel(x, t_emb, W_scale, W_shift):
    b, s, c_per = x.shape
    c_total = c_per * N_DEV
    eps = 1e-5
    rows = s // N_CHUNKS

    def body(x_ref, t_ref, ws_ref, wsh_ref, out_hbm,
             stats_ref, comm_ref, stage_ref, send_sems, recv_sems, copy_sems):
        my = lax.axis_index("i")

        ones_row = jnp.ones((1, c_per), jnp.float32)
        dn = (((1,), (1,)), ((), ()))
        for bi in range(b):
            for k in range(N_CHUNKS):
                r0 = k * rows
                xc = x_ref[bi, r0:r0 + rows, :]
                stats_ref[2 * bi:2 * bi + 1, r0:r0 + rows] = lax.dot_general(
                    ones_row, xc, dn, preferred_element_type=jnp.float32)
                stats_ref[2 * bi + 1:2 * bi + 2, r0:r0 + rows] = lax.dot_general(
                    ones_row, xc * xc, dn, preferred_element_type=jnp.float32)

        bsem = pltpu.get_barrier_semaphore()
        for d in range(1, N_DEV):
            pl.semaphore_signal(
                bsem, inc=1,
                device_id=((my + d) % N_DEV,),
                device_id_type=pl.DeviceIdType.MESH,
            )
        pl.semaphore_wait(bsem, N_DEV - 1)

        rdmas = []
        for d in range(1, N_DEV):
            rdma = pltpu.make_async_remote_copy(
                src_ref=stats_ref,
                dst_ref=comm_ref.at[d - 1],
                send_sem=send_sems.at[d - 1],
                recv_sem=recv_sems.at[d - 1],
                device_id=((my + d) % N_DEV,),
                device_id_type=pl.DeviceIdType.MESH,
            )
            rdma.start()
            rdmas.append(rdma)

        scale = jnp.dot(t_ref[...], ws_ref[...],
                        preferred_element_type=jnp.float32)
        shift = jnp.dot(t_ref[...], wsh_ref[...],
                        preferred_element_type=jnp.float32)

        for r in rdmas:
            r.wait_recv()
        total_row = stats_ref[...] + comm_ref[0] + comm_ref[1] + comm_ref[2]
        total = jnp.transpose(total_row)

        pending = [None, None]
        for bi in range(b):
            mean = total[:, 2 * bi:2 * bi + 1] * (1.0 / c_total)
            var = total[:, 2 * bi + 1:2 * bi + 2] * (1.0 / c_total) - mean * mean
            rstd = lax.rsqrt(var + eps)
            a_col = rstd.astype(jnp.bfloat16)
            b_col = (-mean * rstd).astype(jnp.bfloat16)
            sc = (1.0 + scale[bi:bi + 1, :]).astype(jnp.bfloat16)
            sh = shift[bi:bi + 1, :].astype(jnp.bfloat16)
            for k in range(N_CHUNKS):
                slot = (bi * N_CHUNKS + k) % 2
                if pending[slot] is not None:
                    pending[slot].wait()
                r0 = k * rows
                xc = x_ref[bi, r0:r0 + rows, :].astype(jnp.bfloat16)
                stage_ref[slot] = (xc * a_col[r0:r0 + rows, :]
                                   + b_col[r0:r0 + rows, :]) * sc + sh
                cp = pltpu.make_async_copy(
                    stage_ref.at[slot],
                    out_hbm.at[bi, pl.ds(r0, rows), :],
                    copy_sems.at[slot],
                )
                cp.start()
                pending[slot] = cp
        for cp in pending:
            cp.wait()

        for r in rdmas:
            r.wait_send()

    out_shape = jax.ShapeDtypeStruct((b, s, c_per), jnp.bfloat16)
    return pl.pallas_call(
        body,
        out_shape=out_shape,
        in_specs=[pl.BlockSpec(memory_space=pltpu.VMEM)] * 4,
        out_specs=pl.BlockSpec(memory_space=pl.ANY),
        scratch_shapes=[
            pltpu.VMEM((2 * b, s), jnp.float32),
            pltpu.VMEM((N_DEV - 1, 2 * b, s), jnp.float32),
            pltpu.VMEM((2, rows, c_per), jnp.bfloat16),
            pltpu.SemaphoreType.DMA((N_DEV - 1,)),
            pltpu.SemaphoreType.DMA((N_DEV - 1,)),
            pltpu.SemaphoreType.DMA((2,)),
        ],
        compiler_params=pltpu.CompilerParams(collective_id=0),
    )(x, t_emb, W_scale, W_shift)


# device time: 26501 ns/iter; 1.0823x vs baseline; 1.0823x over previous
import jax
import jax.numpy as jnp
from jax import lax
from jax.experimental import pallas as pl
from jax.experimental.pallas import tpu as pltpu

N_DEV = 4
N_CHUNKS = 8


def kernel(x, t_emb, W_scale, W_shift):
    b, s, c_per = x.shape
    c_total = c_per * N_DEV
    eps = 1e-5
    rows = s // N_CHUNKS

    def body(x_hbm, t_ref, ws_ref, wsh_ref, out_hbm,
             xv_ref, stats_ref, comm_ref, stage_ref,
             load_sems, send_sems, recv_sems, copy_sems):
        my = lax.axis_index("i")

        loads = []
        for bi in range(b):
            for k in range(N_CHUNKS):
                r0 = k * rows
                ld = pltpu.make_async_copy(
                    x_hbm.at[bi, pl.ds(r0, rows), :],
                    xv_ref.at[bi, pl.ds(r0, rows), :],
                    load_sems.at[bi * N_CHUNKS + k],
                )
                ld.start()
                loads.append(ld)

        ones_row = jnp.ones((1, c_per), jnp.float32)
        dn = (((1,), (1,)), ((), ()))
        for bi in range(b):
            for k in range(N_CHUNKS):
                loads[bi * N_CHUNKS + k].wait()
                r0 = k * rows
                xc = xv_ref[bi, r0:r0 + rows, :]
                stats_ref[2 * bi:2 * bi + 1, r0:r0 + rows] = lax.dot_general(
                    ones_row, xc, dn, preferred_element_type=jnp.float32)
                stats_ref[2 * bi + 1:2 * bi + 2, r0:r0 + rows] = lax.dot_general(
                    ones_row, xc * xc, dn, preferred_element_type=jnp.float32)

        bsem = pltpu.get_barrier_semaphore()
        for d in range(1, N_DEV):
            pl.semaphore_signal(
                bsem, inc=1,
                device_id=((my + d) % N_DEV,),
                device_id_type=pl.DeviceIdType.MESH,
            )
        pl.semaphore_wait(bsem, N_DEV - 1)

        rdmas = []
        for d in range(1, N_DEV):
            rdma = pltpu.make_async_remote_copy(
                src_ref=stats_ref,
                dst_ref=comm_ref.at[d - 1],
                send_sem=send_sems.at[d - 1],
                recv_sem=recv_sems.at[d - 1],
                device_id=((my + d) % N_DEV,),
                device_id_type=pl.DeviceIdType.MESH,
            )
            rdma.start()
            rdmas.append(rdma)

        scale = jnp.dot(t_ref[...], ws_ref[...],
                        preferred_element_type=jnp.float32)
        shift = jnp.dot(t_ref[...], wsh_ref[...],
                        preferred_element_type=jnp.float32)

        for r in rdmas:
            r.wait_recv()
        total_row = stats_ref[...] + comm_ref[0] + comm_ref[1] + comm_ref[2]
        total = jnp.transpose(total_row)

        pending = [None, None]
        for bi in range(b):
            mean = total[:, 2 * bi:2 * bi + 1] * (1.0 / c_total)
            var = total[:, 2 * bi + 1:2 * bi + 2] * (1.0 / c_total) - mean * mean
            rstd = lax.rsqrt(var + eps)
            a_col = rstd.astype(jnp.bfloat16)
            b_col = (-mean * rstd).astype(jnp.bfloat16)
            sc = (1.0 + scale[bi:bi + 1, :]).astype(jnp.bfloat16)
            sh = shift[bi:bi + 1, :].astype(jnp.bfloat16)
            for k in range(N_CHUNKS):
                slot = (bi * N_CHUNKS + k) % 2
                if pending[slot] is not None:
                    pending[slot].wait()
                r0 = k * rows
                xc = xv_ref[bi, r0:r0 + rows, :].astype(jnp.bfloat16)
                stage_ref[slot] = (xc * a_col[r0:r0 + rows, :]
                                   + b_col[r0:r0 + rows, :]) * sc + sh
                cp = pltpu.make_async_copy(
                    stage_ref.at[slot],
                    out_hbm.at[bi, pl.ds(r0, rows), :],
                    copy_sems.at[slot],
                )
                cp.start()
                pending[slot] = cp
        for cp in pending:
            cp.wait()

        for r in rdmas:
            r.wait_send()

    out_shape = jax.ShapeDtypeStruct((b, s, c_per), jnp.bfloat16)
    return pl.pallas_call(
        body,
        out_shape=out_shape,
        in_specs=[
            pl.BlockSpec(memory_space=pl.ANY),
            pl.BlockSpec(memory_space=pltpu.VMEM),
            pl.BlockSpec(memory_space=pltpu.VMEM),
            pl.BlockSpec(memory_space=pltpu.VMEM),
        ],
        out_specs=pl.BlockSpec(memory_space=pl.ANY),
        scratch_shapes=[
            pltpu.VMEM((b, s, c_per), jnp.float32),
            pltpu.VMEM((2 * b, s), jnp.float32),
            pltpu.VMEM((N_DEV - 1, 2 * b, s), jnp.float32),
            pltpu.VMEM((2, rows, c_per), jnp.bfloat16),
            pltpu.SemaphoreType.DMA((b * N_CHUNKS,)),
            pltpu.SemaphoreType.DMA((N_DEV - 1,)),
            pltpu.SemaphoreType.DMA((N_DEV - 1,)),
            pltpu.SemaphoreType.DMA((2,)),
        ],
        compiler_params=pltpu.CompilerParams(collective_id=0),
    )(x, t_emb, W_scale, W_shift)


# device time: 25202 ns/iter; 1.1381x vs baseline; 1.0515x over previous
import jax
import jax.numpy as jnp
from jax import lax
from jax.experimental import pallas as pl
from jax.experimental.pallas import tpu as pltpu

N_DEV = 4
N_CHUNKS = 8


def kernel(x, t_emb, W_scale, W_shift):
    b, s, c_per = x.shape
    c_total = c_per * N_DEV
    eps = 1e-5
    rows = s // N_CHUNKS

    def body(x_hbm, t_ref, ws_ref, wsh_ref, out_hbm,
             xv_ref, stats_ref, comm_ref, stage_ref,
             load_sems, send_sems, recv_sems, copy_sems):
        my = lax.axis_index("i")

        loads = []
        for bi in range(b):
            for k in range(N_CHUNKS):
                r0 = k * rows
                ld = pltpu.make_async_copy(
                    x_hbm.at[bi, pl.ds(r0, rows), :],
                    xv_ref.at[bi, pl.ds(r0, rows), :],
                    load_sems.at[bi * N_CHUNKS + k],
                )
                ld.start()
                loads.append(ld)

        bsem = pltpu.get_barrier_semaphore()
        for d in range(1, N_DEV):
            pl.semaphore_signal(
                bsem, inc=1,
                device_id=((my + d) % N_DEV,),
                device_id_type=pl.DeviceIdType.MESH,
            )
        pl.semaphore_wait(bsem, N_DEV - 1)

        ones_row = jnp.ones((1, c_per), jnp.float32)
        dn = (((1,), (1,)), ((), ()))
        rdmas = []
        for bi in range(b):
            for k in range(N_CHUNKS):
                loads[bi * N_CHUNKS + k].wait()
                r0 = k * rows
                xc = xv_ref[bi, r0:r0 + rows, :]
                stats_ref[2 * bi:2 * bi + 1, r0:r0 + rows] = lax.dot_general(
                    ones_row, xc, dn, preferred_element_type=jnp.float32)
                stats_ref[2 * bi + 1:2 * bi + 2, r0:r0 + rows] = lax.dot_general(
                    ones_row, xc * xc, dn, preferred_element_type=jnp.float32)
            for d in range(1, N_DEV):
                rdma = pltpu.make_async_remote_copy(
                    src_ref=stats_ref.at[pl.ds(2 * bi, 2), :],
                    dst_ref=comm_ref.at[d - 1, pl.ds(2 * bi, 2), :],
                    send_sem=send_sems.at[bi * (N_DEV - 1) + d - 1],
                    recv_sem=recv_sems.at[bi * (N_DEV - 1) + d - 1],
                    device_id=((my + d) % N_DEV,),
                    device_id_type=pl.DeviceIdType.MESH,
                )
                rdma.start()
                rdmas.append(rdma)

        scale = jnp.dot(t_ref[...], ws_ref[...],
                        preferred_element_type=jnp.float32)
        shift = jnp.dot(t_ref[...], wsh_ref[...],
                        preferred_element_type=jnp.float32)

        for r in rdmas:
            r.wait_recv()
        total_row = stats_ref[...] + comm_ref[0] + comm_ref[1] + comm_ref[2]
        total = jnp.transpose(total_row)

        pending = [None, None]
        for bi in range(b):
            mean = total[:, 2 * bi:2 * bi + 1] * (1.0 / c_total)
            var = total[:, 2 * bi + 1:2 * bi + 2] * (1.0 / c_total) - mean * mean
            rstd = lax.rsqrt(var + eps)
            a_col = rstd.astype(jnp.bfloat16)
            b_col = (-mean * rstd).astype(jnp.bfloat16)
            sc = (1.0 + scale[bi:bi + 1, :]).astype(jnp.bfloat16)
            sh = shift[bi:bi + 1, :].astype(jnp.bfloat16)
            for k in range(N_CHUNKS):
                slot = (bi * N_CHUNKS + k) % 2
                if pending[slot] is not None:
                    pending[slot].wait()
                r0 = k * rows
                xc = xv_ref[bi, r0:r0 + rows, :].astype(jnp.bfloat16)
                stage_ref[slot] = (xc * a_col[r0:r0 + rows, :]
                                   + b_col[r0:r0 + rows, :]) * sc + sh
                cp = pltpu.make_async_copy(
                    stage_ref.at[slot],
                    out_hbm.at[bi, pl.ds(r0, rows), :],
                    copy_sems.at[slot],
                )
                cp.start()
                pending[slot] = cp
        for cp in pending:
            cp.wait()

        for r in rdmas:
            r.wait_send()

    out_shape = jax.ShapeDtypeStruct((b, s, c_per), jnp.bfloat16)
    return pl.pallas_call(
        body,
        out_shape=out_shape,
        in_specs=[
            pl.BlockSpec(memory_space=pl.ANY),
            pl.BlockSpec(memory_space=pltpu.VMEM),
            pl.BlockSpec(memory_space=pltpu.VMEM),
            pl.BlockSpec(memory_space=pltpu.VMEM),
        ],
        out_specs=pl.BlockSpec(memory_space=pl.ANY),
        scratch_shapes=[
            pltpu.VMEM((b, s, c_per), jnp.float32),
            pltpu.VMEM((2 * b, s), jnp.float32),
            pltpu.VMEM((N_DEV - 1, 2 * b, s), jnp.float32),
            pltpu.VMEM((2, rows, c_per), jnp.bfloat16),
            pltpu.SemaphoreType.DMA((b * N_CHUNKS,)),
            pltpu.SemaphoreType.DMA((b * (N_DEV - 1),)),
            pltpu.SemaphoreType.DMA((b * (N_DEV - 1),)),
            pltpu.SemaphoreType.DMA((2,)),
        ],
        compiler_params=pltpu.CompilerParams(collective_id=0),
    )(x, t_emb, W_scale, W_shift)


# device time: 24613 ns/iter; 1.1653x vs baseline; 1.0239x over previous
import jax
import jax.numpy as jnp
from jax import lax
from jax.experimental import pallas as pl
from jax.experimental.pallas import tpu as pltpu

N_DEV = 4
N_CHUNKS = 8
N_LAND = 4


def kernel(x, t_emb, W_scale, W_shift):
    b, s, c_per = x.shape
    c_total = c_per * N_DEV
    eps = 1e-5
    rows = s // N_CHUNKS
    n_total = b * N_CHUNKS

    def chunk_bk(idx):
        return idx // N_CHUNKS, (idx % N_CHUNKS) * rows

    def body(x_hbm, t_ref, ws_ref, wsh_ref, out_hbm,
             land_ref, xb_ref, stats_ref, comm_ref, stage_ref,
             load_sems, send_sems, recv_sems, copy_sems):
        my = lax.axis_index("i")

        def start_load(idx):
            bi, r0 = chunk_bk(idx)
            ld = pltpu.make_async_copy(
                x_hbm.at[bi, pl.ds(r0, rows), :],
                land_ref.at[idx % N_LAND],
                load_sems.at[idx % N_LAND],
            )
            ld.start()
            return ld

        loads = [start_load(i) for i in range(N_LAND)]

        bsem = pltpu.get_barrier_semaphore()
        for d in range(1, N_DEV):
            pl.semaphore_signal(
                bsem, inc=1,
                device_id=((my + d) % N_DEV,),
                device_id_type=pl.DeviceIdType.MESH,
            )
        pl.semaphore_wait(bsem, N_DEV - 1)

        ones_row = jnp.ones((1, c_per), jnp.bfloat16)
        dn = (((1,), (1,)), ((), ()))
        rdmas = []
        for idx in range(n_total):
            bi, r0 = chunk_bk(idx)
            loads[idx].wait()
            xc = land_ref[idx % N_LAND].astype(jnp.bfloat16)
            xb_ref[bi, r0:r0 + rows, :] = xc
            stats_ref[2 * bi:2 * bi + 1, r0:r0 + rows] = lax.dot_general(
                ones_row, xc, dn, preferred_element_type=jnp.float32)
            stats_ref[2 * bi + 1:2 * bi + 2, r0:r0 + rows] = lax.dot_general(
                ones_row, xc * xc, dn, preferred_element_type=jnp.float32)
            if idx + N_LAND < n_total:
                loads.append(start_load(idx + N_LAND))
            if idx % N_CHUNKS == N_CHUNKS - 1:
                for d in range(1, N_DEV):
                    rdma = pltpu.make_async_remote_copy(
                        src_ref=stats_ref.at[pl.ds(2 * bi, 2), :],
                        dst_ref=comm_ref.at[d - 1, pl.ds(2 * bi, 2), :],
                        send_sem=send_sems.at[bi * (N_DEV - 1) + d - 1],
                        recv_sem=recv_sems.at[bi * (N_DEV - 1) + d - 1],
                        device_id=((my + d) % N_DEV,),
                        device_id_type=pl.DeviceIdType.MESH,
                    )
                    rdma.start()
                    rdmas.append(rdma)

        scale = jnp.dot(t_ref[...], ws_ref[...],
                        preferred_element_type=jnp.float32)
        shift = jnp.dot(t_ref[...], wsh_ref[...],
                        preferred_element_type=jnp.float32)

        for r in rdmas:
            r.wait_recv()
        total_row = stats_ref[...] + comm_ref[0] + comm_ref[1] + comm_ref[2]
        total = jnp.transpose(total_row)

        pending = [None, None]
        for bi in range(b):
            mean = total[:, 2 * bi:2 * bi + 1] * (1.0 / c_total)
            var = total[:, 2 * bi + 1:2 * bi + 2] * (1.0 / c_total) - mean * mean
            rstd = lax.rsqrt(var + eps)
            a_col = rstd.astype(jnp.bfloat16)
            b_col = (-mean * rstd).astype(jnp.bfloat16)
            sc = (1.0 + scale[bi:bi + 1, :]).astype(jnp.bfloat16)
            sh = shift[bi:bi + 1, :].astype(jnp.bfloat16)
            for k in range(N_CHUNKS):
                slot = (bi * N_CHUNKS + k) % 2
                if pending[slot] is not None:
                    pending[slot].wait()
                r0 = k * rows
                xc = xb_ref[bi, r0:r0 + rows, :]
                stage_ref[slot] = (xc * a_col[r0:r0 + rows, :]
                                   + b_col[r0:r0 + rows, :]) * sc + sh
                cp = pltpu.make_async_copy(
                    stage_ref.at[slot],
                    out_hbm.at[bi, pl.ds(r0, rows), :],
                    copy_sems.at[slot],
                )
                cp.start()
                pending[slot] = cp
        for cp in pending:
            cp.wait()

        for r in rdmas:
            r.wait_send()

    out_shape = jax.ShapeDtypeStruct((b, s, c_per), jnp.bfloat16)
    return pl.pallas_call(
        body,
        out_shape=out_shape,
        in_specs=[
            pl.BlockSpec(memory_space=pl.ANY),
            pl.BlockSpec(memory_space=pltpu.VMEM),
            pl.BlockSpec(memory_space=pltpu.VMEM),
            pl.BlockSpec(memory_space=pltpu.VMEM),
        ],
        out_specs=pl.BlockSpec(memory_space=pl.ANY),
        scratch_shapes=[
            pltpu.VMEM((N_LAND, rows, c_per), jnp.float32),
            pltpu.VMEM((b, s, c_per), jnp.bfloat16),
            pltpu.VMEM((2 * b, s), jnp.float32),
            pltpu.VMEM((N_DEV - 1, 2 * b, s), jnp.float32),
            pltpu.VMEM((2, rows, c_per), jnp.bfloat16),
            pltpu.SemaphoreType.DMA((N_LAND,)),
            pltpu.SemaphoreType.DMA((b * (N_DEV - 1),)),
            pltpu.SemaphoreType.DMA((b * (N_DEV - 1),)),
            pltpu.SemaphoreType.DMA((2,)),
        ],
        compiler_params=pltpu.CompilerParams(collective_id=0),
    )(x, t_emb, W_scale, W_shift)


# device time: 24583 ns/iter; 1.1667x vs baseline; 1.0012x over previous
import jax
import jax.numpy as jnp
from jax import lax
from jax.experimental import pallas as pl
from jax.experimental.pallas import tpu as pltpu

N_DEV = 4
N_CHUNKS = 8
N_LAND = 8


def kernel(x, t_emb, W_scale, W_shift):
    b, s, c_per = x.shape
    c_total = c_per * N_DEV
    eps = 1e-5
    rows = s // N_CHUNKS
    n_total = b * N_CHUNKS

    def chunk_bk(idx):
        return idx // N_CHUNKS, (idx % N_CHUNKS) * rows

    def body(x_hbm, t_ref, ws_ref, wsh_ref, out_hbm,
             land_ref, xb_ref, stats_ref, comm_ref, stage_ref,
             load_sems, send_sems, recv_sems, copy_sems):
        my = lax.axis_index("i")

        def start_load(idx):
            bi, r0 = chunk_bk(idx)
            ld = pltpu.make_async_copy(
                x_hbm.at[bi, pl.ds(r0, rows), :],
                land_ref.at[idx % N_LAND],
                load_sems.at[idx % N_LAND],
            )
            ld.start()
            return ld

        loads = [start_load(i) for i in range(N_LAND)]

        bsem = pltpu.get_barrier_semaphore()
        for d in range(1, N_DEV):
            pl.semaphore_signal(
                bsem, inc=1,
                device_id=((my + d) % N_DEV,),
                device_id_type=pl.DeviceIdType.MESH,
            )
        pl.semaphore_wait(bsem, N_DEV - 1)

        ones_row = jnp.ones((1, c_per), jnp.bfloat16)
        dn = (((1,), (1,)), ((), ()))
        rdmas = []
        for idx in range(n_total):
            bi, r0 = chunk_bk(idx)
            loads[idx].wait()
            xc = land_ref[idx % N_LAND].astype(jnp.bfloat16)
            xb_ref[bi, r0:r0 + rows, :] = xc
            stats_ref[2 * bi:2 * bi + 1, r0:r0 + rows] = lax.dot_general(
                ones_row, xc, dn, preferred_element_type=jnp.float32)
            stats_ref[2 * bi + 1:2 * bi + 2, r0:r0 + rows] = lax.dot_general(
                ones_row, xc * xc, dn, preferred_element_type=jnp.float32)
            if idx + N_LAND < n_total:
                loads.append(start_load(idx + N_LAND))
            if idx % N_CHUNKS == N_CHUNKS - 1:
                for d in range(1, N_DEV):
                    rdma = pltpu.make_async_remote_copy(
                        src_ref=stats_ref.at[pl.ds(2 * bi, 2), :],
                        dst_ref=comm_ref.at[d - 1, pl.ds(2 * bi, 2), :],
                        send_sem=send_sems.at[bi * (N_DEV - 1) + d - 1],
                        recv_sem=recv_sems.at[bi * (N_DEV - 1) + d - 1],
                        device_id=((my + d) % N_DEV,),
                        device_id_type=pl.DeviceIdType.MESH,
                    )
                    rdma.start()
                    rdmas.append(rdma)

        scale = jnp.dot(t_ref[...], ws_ref[...],
                        preferred_element_type=jnp.float32)
        shift = jnp.dot(t_ref[...], wsh_ref[...],
                        preferred_element_type=jnp.float32)

        for r in rdmas:
            r.wait_recv()
        total_row = stats_ref[...] + comm_ref[0] + comm_ref[1] + comm_ref[2]
        total = jnp.transpose(total_row)

        pending = [None, None]
        for bi in range(b):
            mean = total[:, 2 * bi:2 * bi + 1] * (1.0 / c_total)
            var = total[:, 2 * bi + 1:2 * bi + 2] * (1.0 / c_total) - mean * mean
            rstd = lax.rsqrt(var + eps)
            a_col = rstd.astype(jnp.bfloat16)
            b_col = (-mean * rstd).astype(jnp.bfloat16)
            sc = (1.0 + scale[bi:bi + 1, :]).astype(jnp.bfloat16)
            sh = shift[bi:bi + 1, :].astype(jnp.bfloat16)
            for k in range(N_CHUNKS):
                slot = (bi * N_CHUNKS + k) % 2
                if pending[slot] is not None:
                    pending[slot].wait()
                r0 = k * rows
                xc = xb_ref[bi, r0:r0 + rows, :]
                stage_ref[slot] = (xc * a_col[r0:r0 + rows, :]
                                   + b_col[r0:r0 + rows, :]) * sc + sh
                cp = pltpu.make_async_copy(
                    stage_ref.at[slot],
                    out_hbm.at[bi, pl.ds(r0, rows), :],
                    copy_sems.at[slot],
                )
                cp.start()
                pending[slot] = cp
        for cp in pending:
            cp.wait()

        for r in rdmas:
            r.wait_send()

    out_shape = jax.ShapeDtypeStruct((b, s, c_per), jnp.bfloat16)
    return pl.pallas_call(
        body,
        out_shape=out_shape,
        in_specs=[
            pl.BlockSpec(memory_space=pl.ANY),
            pl.BlockSpec(memory_space=pltpu.VMEM),
            pl.BlockSpec(memory_space=pltpu.VMEM),
            pl.BlockSpec(memory_space=pltpu.VMEM),
        ],
        out_specs=pl.BlockSpec(memory_space=pl.ANY),
        scratch_shapes=[
            pltpu.VMEM((N_LAND, rows, c_per), jnp.float32),
            pltpu.VMEM((b, s, c_per), jnp.bfloat16),
            pltpu.VMEM((2 * b, s), jnp.float32),
            pltpu.VMEM((N_DEV - 1, 2 * b, s), jnp.float32),
            pltpu.VMEM((2, rows, c_per), jnp.bfloat16),
            pltpu.SemaphoreType.DMA((N_LAND,)),
            pltpu.SemaphoreType.DMA((b * (N_DEV - 1),)),
            pltpu.SemaphoreType.DMA((b * (N_DEV - 1),)),
            pltpu.SemaphoreType.DMA((2,)),
        ],
        compiler_params=pltpu.CompilerParams(collective_id=0),
    )(x, t_emb, W_scale, W_shift)
